# Initial kernel scaffold; baseline (speedup 1.0000x reference)
import jax
import jax.numpy as jnp
from jax.experimental import pallas as pl


def kernel(x, x_len, codebooks):
    raise NotImplementedError("write your pallas kernel here")


# speedup vs baseline: 13.9559x; 13.9559x over previous
import functools

import jax
import jax.numpy as jnp
from jax import lax
from jax.experimental import pallas as pl
from jax.experimental.pallas import tpu as pltpu
from jax.experimental.pallas import tpu_sc as plsc

B, Q, T, K, D = 16, 8, 2048, 1024, 128
NC, NS, L = 2, 16, 16
NW = NC * NS
TPW = (B * T) // NW
C = 64
NCHUNK = TPW // C
ZROW = Q * K
VPR = D // L

_mesh = plsc.VectorSubcoreMesh(core_axis_name="c", subcore_axis_name="s")


@functools.partial(
    pl.kernel,
    out_type=jax.ShapeDtypeStruct((B, T, D), jnp.float32),
    mesh=_mesh,
    scratch_types=[
        pltpu.VMEM((Q, TPW), jnp.int32),
        pltpu.VMEM((Q, TPW), jnp.int32),
        pltpu.VMEM((Q, C, D), jnp.float32),
        pltpu.VMEM((C, D), jnp.float32),
        pltpu.VMEM((C, D), jnp.float32),
        pltpu.VMEM((L,), jnp.int32),
        pltpu.SemaphoreType.DMA,
    ],
)
def _dequant(x_hbm, xlen_hbm, tab_hbm, out_hbm,
             xbuf, idxbuf, tmp, obuf, zbuf, lenbuf, gsem):
    cid = lax.axis_index("c")
    sid = lax.axis_index("s")
    wid = sid * NC + cid
    b = wid // 2
    t_base = (wid % 2) * TPW

    pltpu.sync_copy(xlen_hbm.at[b], lenbuf)
    for q in range(Q):
        pltpu.sync_copy(x_hbm.at[b, q, pl.ds(t_base, TPW)], xbuf.at[q])

    iota = lax.iota(jnp.int32, L)
    lenv = lenbuf[...]
    slen = lenv[0]

    def idx_body(j, _):
        tv = (t_base + j * L) + iota
        m = tv < lenv
        for q in range(Q):
            xv = xbuf[q, pl.ds(j * L, L)]
            idxbuf[q, pl.ds(j * L, L)] = jnp.where(m, xv + q * K, ZROW)
        return 0

    lax.fori_loop(0, TPW // L, idx_body, 0)

    def zero_body(r, _):
        for v in range(VPR):
            zbuf[r, pl.ds(v * L, L)] = jnp.zeros((L,), jnp.float32)
        return 0

    lax.fori_loop(0, C, zero_body, 0)

    def chunk_body(ci, _):
        t0 = t_base + ci * C

        @pl.when(t0 < slen)
        def _active():
            cps = [
                pltpu.async_copy(tab_hbm.at[idxbuf.at[q, pl.ds(ci * C, C)]],
                                 tmp.at[q], gsem)
                for q in range(Q)
            ]
            for cp in cps:
                cp.wait()

            def acc_body(r, _):
                for v in range(VPR):
                    s = tmp[0, r, pl.ds(v * L, L)]
                    for q in range(1, Q):
                        s = s + tmp[q, r, pl.ds(v * L, L)]
                    obuf[r, pl.ds(v * L, L)] = s
                return 0

            lax.fori_loop(0, C, acc_body, 0)
            pltpu.sync_copy(obuf, out_hbm.at[b, pl.ds(t0, C)])

        @pl.when(t0 >= slen)
        def _masked():
            pltpu.sync_copy(zbuf, out_hbm.at[b, pl.ds(t0, C)])

        return 0

    lax.fori_loop(0, NCHUNK, chunk_body, 0)


def kernel(x, x_len, codebooks):
    tab = jnp.concatenate(
        [codebooks.reshape(Q * K, D), jnp.zeros((8, D), jnp.float32)], axis=0)
    xlen_b = jnp.broadcast_to(x_len[:, None], (B, L))
    return _dequant(x, xlen_b, tab)


# speedup vs baseline: 14.0782x; 1.0088x over previous
import functools

import jax
import jax.numpy as jnp
from jax import lax
from jax.experimental import pallas as pl
from jax.experimental.pallas import tpu as pltpu
from jax.experimental.pallas import tpu_sc as plsc

B, Q, T, K, D = 16, 8, 2048, 1024, 128
NC, NS, L = 2, 16, 16
NW = NC * NS
TPW = (B * T) // NW
C = 64
NCHUNK = TPW // C
ZROW = Q * K
VPR = D // L

_mesh = plsc.VectorSubcoreMesh(core_axis_name="c", subcore_axis_name="s")


@functools.partial(
    pl.kernel,
    out_type=jax.ShapeDtypeStruct((B, T, D), jnp.float32),
    mesh=_mesh,
    scratch_types=[
        pltpu.VMEM((Q, TPW), jnp.int32),
        pltpu.VMEM((Q, TPW), jnp.int32),
        pltpu.VMEM((C, D), jnp.float32),
        pltpu.VMEM((C, D), jnp.float32),
        pltpu.VMEM((L,), jnp.int32),
        pltpu.SemaphoreType.DMA,
    ],
)
def _dequant(x_hbm, xlen_hbm, tab_hbm, out_hbm,
             xbuf, idxbuf, acc, zbuf, lenbuf, gsem):
    cid = lax.axis_index("c")
    sid = lax.axis_index("s")
    wid = sid * NC + cid
    b = wid // 2
    t_base = (wid % 2) * TPW

    pltpu.sync_copy(xlen_hbm.at[b], lenbuf)
    for q in range(Q):
        pltpu.sync_copy(x_hbm.at[b, q, pl.ds(t_base, TPW)], xbuf.at[q])

    iota = lax.iota(jnp.int32, L)
    lenv = lenbuf[...]
    slen = lenv[0]

    def idx_body(j, _):
        tv = (t_base + j * L) + iota
        m = tv < lenv
        for q in range(Q):
            xv = xbuf[q, pl.ds(j * L, L)]
            idxbuf[q, pl.ds(j * L, L)] = jnp.where(m, xv + q * K, ZROW)
        return 0

    lax.fori_loop(0, TPW // L, idx_body, 0)

    def zero_body(r, _):
        for v in range(VPR):
            zbuf[r, pl.ds(v * L, L)] = jnp.zeros((L,), jnp.float32)
        return 0

    lax.fori_loop(0, C, zero_body, 0)

    def chunk_body(ci, _):
        t0 = t_base + ci * C

        @pl.when(t0 < slen)
        def _active():
            def zero_acc(r, _):
                for v in range(VPR):
                    acc[r, pl.ds(v * L, L)] = jnp.zeros((L,), jnp.float32)
                return 0

            lax.fori_loop(0, C, zero_acc, 0)
            cps = [
                pltpu.async_copy(tab_hbm.at[idxbuf.at[q, pl.ds(ci * C, C)]],
                                 acc, gsem, add=True)
                for q in range(Q)
            ]
            for cp in cps:
                cp.wait()
            pltpu.sync_copy(acc, out_hbm.at[b, pl.ds(t0, C)])

        @pl.when(t0 >= slen)
        def _masked():
            pltpu.sync_copy(zbuf, out_hbm.at[b, pl.ds(t0, C)])

        return 0

    lax.fori_loop(0, NCHUNK, chunk_body, 0)


def kernel(x, x_len, codebooks):
    tab = jnp.concatenate(
        [codebooks.reshape(Q * K, D), jnp.zeros((8, D), jnp.float32)], axis=0)
    xlen_b = jnp.broadcast_to(x_len[:, None], (B, L))
    return _dequant(x, xlen_b, tab)


# speedup vs baseline: 41.2501x; 2.9301x over previous
import functools

import jax
import jax.numpy as jnp
from jax import lax
from jax.experimental import pallas as pl
from jax.experimental.pallas import tpu as pltpu
from jax.experimental.pallas import tpu_sc as plsc

B, Q, T, K, D = 16, 8, 2048, 1024, 128
NC, NS, L = 2, 16, 16
NW = NC * NS
TPW = (B * T) // NW
C = 64
NCHUNK = TPW // C
ZROW = Q * K
VPR = D // L
NTAB = Q * K + 128
RPS = NTAB // NS

_mesh = plsc.VectorSubcoreMesh(core_axis_name="c", subcore_axis_name="s")


@functools.partial(
    pl.kernel,
    out_type=jax.ShapeDtypeStruct((B, T, D), jnp.float32),
    mesh=_mesh,
    scratch_types=[
        pltpu.VMEM((Q, TPW), jnp.int32),
        pltpu.VMEM((Q, TPW), jnp.int32),
        pltpu.VMEM((C, D), jnp.float32),
        pltpu.VMEM((C, D), jnp.float32),
        pltpu.VMEM((L,), jnp.int32),
        pltpu.VMEM_SHARED((NTAB, D), jnp.float32),
        pltpu.SemaphoreType.DMA,
    ],
)
def _dequant(x_hbm, xlen_hbm, tab_hbm, out_hbm,
             xbuf, idxbuf, acc, zbuf, lenbuf, stab, gsem):
    cid = lax.axis_index("c")
    sid = lax.axis_index("s")
    wid = sid * NC + cid
    b = wid // 2
    t_base = (wid % 2) * TPW

    pltpu.sync_copy(tab_hbm.at[pl.ds(sid * RPS, RPS)],
                    stab.at[pl.ds(sid * RPS, RPS)])
    pltpu.sync_copy(xlen_hbm.at[b], lenbuf)
    for q in range(Q):
        pltpu.sync_copy(x_hbm.at[b, q, pl.ds(t_base, TPW)], xbuf.at[q])

    iota = lax.iota(jnp.int32, L)
    lenv = lenbuf[...]
    slen = lenv[0]

    def idx_body(j, _):
        tv = (t_base + j * L) + iota
        m = tv < lenv
        for q in range(Q):
            xv = xbuf[q, pl.ds(j * L, L)]
            idxbuf[q, pl.ds(j * L, L)] = jnp.where(m, xv + q * K, ZROW)
        return 0

    lax.fori_loop(0, TPW // L, idx_body, 0)

    def zero_body(r, _):
        for v in range(VPR):
            zbuf[r, pl.ds(v * L, L)] = jnp.zeros((L,), jnp.float32)
        return 0

    lax.fori_loop(0, C, zero_body, 0)
    plsc.subcore_barrier()

    def chunk_body(ci, _):
        t0 = t_base + ci * C

        @pl.when(t0 < slen)
        def _active():
            def zero_acc(r, _):
                for v in range(VPR):
                    acc[r, pl.ds(v * L, L)] = jnp.zeros((L,), jnp.float32)
                return 0

            lax.fori_loop(0, C, zero_acc, 0)
            cps = [
                pltpu.async_copy(stab.at[idxbuf.at[q, pl.ds(ci * C, C)]],
                                 acc, gsem, add=True)
                for q in range(Q)
            ]
            for cp in cps:
                cp.wait()
            pltpu.sync_copy(acc, out_hbm.at[b, pl.ds(t0, C)])

        @pl.when(t0 >= slen)
        def _masked():
            pltpu.sync_copy(zbuf, out_hbm.at[b, pl.ds(t0, C)])

        return 0

    lax.fori_loop(0, NCHUNK, chunk_body, 0)


def kernel(x, x_len, codebooks):
    tab = jnp.concatenate(
        [codebooks.reshape(Q * K, D),
         jnp.zeros((NTAB - Q * K, D), jnp.float32)], axis=0)
    xlen_b = jnp.broadcast_to(x_len[:, None], (B, L))
    return _dequant(x, xlen_b, tab)


# speedup vs baseline: 54.2615x; 1.3154x over previous
import functools

import jax
import jax.numpy as jnp
from jax import lax
from jax.experimental import pallas as pl
from jax.experimental.pallas import tpu as pltpu
from jax.experimental.pallas import tpu_sc as plsc

B, Q, T, K, D = 16, 8, 2048, 1024, 128
NC, NS, L = 2, 16, 16
NW = NC * NS
C = 64
NCH = T // C
NIT = B * NCH // NW
STRIDE = 5
ZROW = Q * K
VPR = D // L
NTAB = Q * K + 128
RPS = NTAB // NS

_mesh = plsc.VectorSubcoreMesh(core_axis_name="c", subcore_axis_name="s")


@functools.partial(
    pl.kernel,
    out_type=jax.ShapeDtypeStruct((B, T, D), jnp.float32),
    mesh=_mesh,
    scratch_types=[
        pltpu.VMEM((NIT, Q, C), jnp.int32),
        pltpu.VMEM((NIT, Q, C), jnp.int32),
        pltpu.VMEM((2, C, D), jnp.float32),
        pltpu.VMEM((L,), jnp.int32),
        pltpu.VMEM_SHARED((NTAB, D), jnp.float32),
        pltpu.SemaphoreType.DMA,
        pltpu.SemaphoreType.DMA,
        pltpu.SemaphoreType.DMA,
        pltpu.SemaphoreType.DMA,
        pltpu.SemaphoreType.DMA,
        pltpu.SemaphoreType.DMA,
    ],
)
def _dequant(x_hbm, xlen_hbm, tab_hbm, out_hbm,
             xbuf, idxbuf, acc, lenbuf, stab,
             tsem, xsem, gsem0, gsem1, osem0, osem1):
    cid = lax.axis_index("c")
    sid = lax.axis_index("s")
    wid = sid * NC + cid
    gsem = (gsem0, gsem1)
    osem = (osem0, osem1)

    tcp = pltpu.async_copy(tab_hbm.at[pl.ds(sid * RPS, RPS)],
                           stab.at[pl.ds(sid * RPS, RPS)], tsem)
    pltpu.sync_copy(xlen_hbm, lenbuf)
    cof = []
    xcps = []
    for k in range(NIT):
        ck = lax.rem(wid + STRIDE * k, NCH)
        cof.append(ck * C)
        xcps.append([pltpu.async_copy(x_hbm.at[k, q, pl.ds(cof[k], C)],
                                      xbuf.at[k, q], xsem) for q in range(Q)])

    iota = lax.iota(jnp.int32, L)
    lv = lenbuf[...]

    active = []
    for k in range(NIT):
        for cp in xcps[k]:
            cp.wait()
        lenk = jnp.full((L,), lv[k], jnp.int32)
        active.append(cof[k] < lv[k])

        def idx_body(j, _, k=k, lenk=lenk):
            tv = (cof[k] + j * L) + iota
            m = tv < lenk
            for q in range(Q):
                xv = xbuf[k, q, pl.ds(j * L, L)]
                idxbuf[k, q, pl.ds(j * L, L)] = jnp.where(m, xv + q * K, ZROW)
            return 0

        lax.fori_loop(0, C // L, idx_body, 0)

    tcp.wait()
    plsc.subcore_barrier()

    def zero_acc(p):
        def body(r, _):
            for v in range(VPR):
                acc[p, r, pl.ds(v * L, L)] = jnp.zeros((L,), jnp.float32)
            return 0
        lax.fori_loop(0, C, body, 0)

    def fire_gathers(k, p):
        @pl.when(active[k])
        def _():
            for q in range(Q):
                pltpu.async_copy(stab.at[idxbuf.at[k, q]], acc.at[p],
                                 gsem[p], add=True)

    def drain_gathers(k, p):
        @pl.when(active[k])
        def _():
            for q in range(Q):
                pltpu.make_async_copy(stab.at[idxbuf.at[k, q]], acc.at[p],
                                      gsem[p]).wait()

    def fire_out(k, p):
        pltpu.async_copy(acc.at[p], out_hbm.at[k, pl.ds(cof[k], C)], osem[p])

    def drain_out(k, p):
        pltpu.make_async_copy(acc.at[p], out_hbm.at[k, pl.ds(cof[k], C)],
                              osem[p]).wait()

    zero_acc(0)
    fire_gathers(0, 0)
    for k in range(1, NIT):
        p, pp = k % 2, (k - 1) % 2
        if k >= 2:
            drain_out(k - 2, p)
        zero_acc(p)
        fire_gathers(k, p)
        drain_gathers(k - 1, pp)
        fire_out(k - 1, pp)
    drain_gathers(NIT - 1, (NIT - 1) % 2)
    fire_out(NIT - 1, (NIT - 1) % 2)
    drain_out(NIT - 2, NIT % 2)
    drain_out(NIT - 1, (NIT - 1) % 2)


def kernel(x, x_len, codebooks):
    tab = jnp.concatenate(
        [codebooks.reshape(Q * K, D),
         jnp.zeros((NTAB - Q * K, D), jnp.float32)], axis=0)
    return _dequant(x, x_len, tab)
